# Initial kernel scaffold; baseline (speedup 1.0000x reference)
import jax
import jax.numpy as jnp
from jax.experimental import pallas as pl


def kernel(x, misc, tiles, coord, piece, row, col, tilecolor, zeros, bias):
    raise NotImplementedError("write your pallas kernel here")


# speedup vs baseline: 2.0582x; 2.0582x over previous
import functools

import jax
import jax.numpy as jnp
import numpy as np
from jax import lax
from jax.experimental import pallas as pl
from jax.experimental.pallas import tpu as pltpu
from jax.experimental.pallas import tpu_sc as plsc

DOUT = 64
BATCH = 16384
L = 50
LPAD = 56
VOCAB = 777
TROWS = 800
ZROW = 776

NC = 2
NS = 16
NW = NC * NS

BW = BATCH // NW
CB = 2
CHT = CB * LPAD
NCH = BW // CB
TOKW = BW * LPAD


def _white_mask_np():
    m = np.zeros((1, 8, 8, 1), dtype=np.float32)
    for y in range(8):
        for x in range(8):
            m[0, y, x, 0] = float((y + x) % 2 == 0)
    return m


_MASK_NP = np.broadcast_to(_white_mask_np(), (1, 8, 8, DOUT)).copy()


def _table_body(tiles_ref, coord_ref, piece_ref, row_ref, col_ref, tc_ref,
                mask_ref, o_ref):
    o_ref[...] = (tiles_ref[...] + coord_ref[...] + piece_ref[...]
                  + row_ref[...] + col_ref[...]
                  + tc_ref[...] * mask_ref[...])


def _build_table(tiles, coord, piece, row, col, tilecolor):
    return pl.pallas_call(
        _table_body,
        out_shape=jax.ShapeDtypeStruct((12, 8, 8, DOUT), jnp.float32),
    )(tiles, coord, piece, row, col, tilecolor, jnp.asarray(_MASK_NP))


def _sc_body(xp_hbm, table_hbm, bias_hbm, out_hbm,
             x_v, r0, r1, acc_v, bias_v, sem0, sem1):
    wid = lax.axis_index("s") * NC + lax.axis_index("c")

    pltpu.sync_copy(bias_hbm, bias_v)
    pltpu.sync_copy(xp_hbm.at[pl.ds(wid * TOKW, TOKW)], x_v)

    def fire(g, buf, sem):
        idx = x_v.at[pl.ds(g * CHT, CHT)]
        pltpu.async_copy(table_hbm.at[idx], buf, sem)

    def wait(buf, sem):
        idx = x_v.at[pl.ds(0, CHT)]
        pltpu.make_async_copy(table_hbm.at[idx], buf, sem).wait()

    fire(0, r0, sem0)
    fire(1, r1, sem1)

    def process(g, buf):
        for r in range(CB):
            def lbody(l, carry):
                a0, a1, a2, a3 = carry
                t = r * LPAD + l
                return (a0 + buf[t, pl.ds(0, 16)],
                        a1 + buf[t, pl.ds(16, 16)],
                        a2 + buf[t, pl.ds(32, 16)],
                        a3 + buf[t, pl.ds(48, 16)])
            init = (bias_v[pl.ds(0, 16)], bias_v[pl.ds(16, 16)],
                    bias_v[pl.ds(32, 16)], bias_v[pl.ds(48, 16)])
            a0, a1, a2, a3 = lax.fori_loop(0, L, lbody, init)
            lr = g * CB + r
            acc_v[lr, pl.ds(0, 16)] = a0
            acc_v[lr, pl.ds(16, 16)] = a1
            acc_v[lr, pl.ds(32, 16)] = a2
            acc_v[lr, pl.ds(48, 16)] = a3

    def outer(gg, carry):
        g0 = gg * 2
        g1 = gg * 2 + 1
        wait(r0, sem0)
        process(g0, r0)

        @pl.when(g0 + 2 < NCH)
        def _():
            fire(g0 + 2, r0, sem0)

        wait(r1, sem1)
        process(g1, r1)

        @pl.when(g1 + 2 < NCH)
        def _():
            fire(g1 + 2, r1, sem1)

        return carry

    lax.fori_loop(0, NCH // 2, outer, 0)

    pltpu.sync_copy(acc_v, out_hbm.at[pl.ds(wid * BW, BW)])


_sc_emb = functools.partial(
    pl.kernel,
    out_type=jax.ShapeDtypeStruct((BATCH, DOUT), jnp.float32),
    mesh=plsc.VectorSubcoreMesh(core_axis_name="c", subcore_axis_name="s"),
    compiler_params=pltpu.CompilerParams(use_tc_tiling_on_sc=False),
    scratch_types=[
        pltpu.VMEM((TOKW,), jnp.int32),
        pltpu.VMEM((CHT, DOUT), jnp.float32),
        pltpu.VMEM((CHT, DOUT), jnp.float32),
        pltpu.VMEM((BW, DOUT), jnp.float32),
        pltpu.VMEM((DOUT,), jnp.float32),
        pltpu.SemaphoreType.DMA,
        pltpu.SemaphoreType.DMA,
    ],
)(_sc_body)


def kernel(x, misc, tiles, coord, piece, row, col, tilecolor, zeros, bias):
    w4 = _build_table(tiles, coord, piece, row, col, tilecolor)
    table = jnp.concatenate(
        [w4.reshape(768, DOUT), misc,
         jnp.zeros((TROWS - 768 - 8, DOUT), jnp.float32)], axis=0)
    xp = jnp.pad(x, ((0, 0), (0, LPAD - L)), constant_values=ZROW)
    return _sc_emb(xp.reshape(-1), table, bias)
